# baseline (speedup 1.0000x reference)
import functools

import jax
import jax.numpy as jnp
from jax import lax
from jax.experimental import pallas as pl
from jax.experimental.pallas import tpu as pltpu
from jax.experimental.pallas import tpu_sc as plsc

_N_BINS = 8
_P = 1024
_D = 64
_G = 64
_T = 4608
_SLICES = (512, 2048, 2048)
_NW = 32
_CW = 16
_TILE = 512


def _onehot_cat(v, edges_ref):
    gt = [(v > edges_ref[i]).astype(jnp.bfloat16) for i in range(7)]
    ohs = [1.0 - gt[0]]
    for b in range(1, 7):
        ohs.append(gt[b - 1] - gt[b])
    ohs.append(gt[6])
    return jnp.concatenate(ohs, axis=1)


def _match_kernel(edges_ref, x_ref, pat_ref, val_ref, base, ts):
    poh = _onehot_cat(pat_ref[...], edges_ref)
    B, S, D = x_ref.shape
    xb = lax.slice(x_ref[...].reshape(B * S, D), (base, 0), (base + ts, D))
    a_cat = _onehot_cat(xb, edges_ref)
    acc = lax.dot_general(
        poh, a_cat, (((1,), (1,)), ((), ())),
        preferred_element_type=jnp.float32)

    iot = lax.broadcasted_iota(jnp.int32, (_P, 1), 0)
    rf = ((_P - 1) - iot).astype(jnp.float32) * (1.0 / _P)
    val = acc + rf
    gmax = jnp.max(val.reshape(_G, 16, val.shape[1]), axis=1)
    val_ref[...] = gmax.T


def _sc_argmax(val_hbm, best_hbm, score_hbm, buf, tmp, bb, sb, *, tpw):
    wid = lax.axis_index("s") * 2 + lax.axis_index("c")
    tbase = wid * tpw
    lane = lax.iota(jnp.int32, 16)

    def chunk_body(c, _):
        pltpu.sync_copy(val_hbm.at[pl.ds(tbase + c * _CW, _CW), :], buf)

        def tok_body(t, m_all):
            m16 = buf[t, pl.ds(0, 16)]
            for g in range(1, _G // 16):
                m16 = jnp.maximum(m16, buf[t, pl.ds(g * 16, 16)])
            for sh in (8, 4, 2, 1):
                tmp[pl.ds(0, 16)] = m16
                tmp[pl.ds(16, 16)] = m16
                m16 = jnp.maximum(m16, tmp[pl.ds(sh, 16)])
            return jnp.where(lane == t, m16, m_all)

        m_all = lax.fori_loop(0, _CW, tok_body,
                              jnp.full((16,), -1.0, jnp.float32))
        cnt = m_all.astype(jnp.int32)
        cf = cnt.astype(jnp.float32)
        k = ((m_all - cf) * float(_P)).astype(jnp.int32)
        bb[pl.ds(c * _CW, _CW)] = (_P - 1) - k
        sb[pl.ds(c * _CW, _CW)] = cf * (1.0 / _D)
        return 0

    lax.fori_loop(0, tpw // _CW, chunk_body, 0)
    pltpu.sync_copy(bb, best_hbm.at[pl.ds(tbase, tpw)])
    pltpu.sync_copy(sb, score_hbm.at[pl.ds(tbase, tpw)])


@functools.lru_cache(maxsize=None)
def _make_sc_argmax(ts):
    tpw = ts // _NW

    @functools.partial(
        pl.kernel,
        mesh=plsc.VectorSubcoreMesh(core_axis_name="c", subcore_axis_name="s"),
        out_type=[
            jax.ShapeDtypeStruct((ts,), jnp.int32),
            jax.ShapeDtypeStruct((ts,), jnp.float32),
        ],
        scratch_types=[
            pltpu.VMEM((_CW, _G), jnp.float32),
            pltpu.VMEM((32,), jnp.float32),
            pltpu.VMEM((tpw,), jnp.int32),
            pltpu.VMEM((tpw,), jnp.float32),
        ],
    )
    def _call(val_hbm, best_hbm, score_hbm, buf, tmp, bb, sb):
        _sc_argmax(val_hbm, best_hbm, score_hbm, buf, tmp, bb, sb, tpw=tpw)

    return _call


def kernel(x, patterns, quantize_edges):
    B, S, D = x.shape
    bests, scores = [], []
    base = 0
    for ts in _SLICES:
        val = pl.pallas_call(
            functools.partial(_match_kernel, base=base, ts=ts),
            in_specs=[
                pl.BlockSpec(memory_space=pltpu.SMEM),
                pl.BlockSpec((B, S, D), lambda: (0, 0, 0)),
                pl.BlockSpec((_P, D), lambda: (0, 0)),
            ],
            out_specs=pl.BlockSpec((ts, _G), lambda: (0, 0)),
            out_shape=jax.ShapeDtypeStruct((ts, _G), jnp.float32),
        )(quantize_edges, x, patterns)
        b1, s1 = _make_sc_argmax(ts)(val)
        bests.append(b1)
        scores.append(s1)
        base += ts
    best = jnp.concatenate(bests)
    score = jnp.concatenate(scores)
    return best.reshape(B, S), score.reshape(B, S)


# speedup vs baseline: 1.0324x; 1.0324x over previous
import functools

import jax
import jax.numpy as jnp
from jax import lax
from jax.experimental import pallas as pl
from jax.experimental.pallas import tpu as pltpu
from jax.experimental.pallas import tpu_sc as plsc

_N_BINS = 8
_P = 1024
_D = 64
_G = 64
_T = 4608
_SLICES = (2048, 2048, 512)
_NW = 32
_CW = 16
_TILE = 512


def _onehot_cat(v, edges_ref):
    gt = [(v > edges_ref[i]).astype(jnp.bfloat16) for i in range(7)]
    ohs = [1.0 - gt[0]]
    for b in range(1, 7):
        ohs.append(gt[b - 1] - gt[b])
    ohs.append(gt[6])
    return jnp.concatenate(ohs, axis=1)


def _match_kernel(edges_ref, x_ref, pat_ref, val_ref, poh_ref, rf_ref):
    @pl.when(pl.program_id(0) == 0)
    def _init():
        poh_ref[...] = _onehot_cat(pat_ref[...], edges_ref)
        iot = lax.broadcasted_iota(jnp.int32, (_P, 1), 0)
        rf_ref[...] = ((_P - 1) - iot).astype(jnp.float32) * (1.0 / _P)

    a_cat = _onehot_cat(x_ref[...], edges_ref)
    acc = lax.dot_general(
        poh_ref[...], a_cat, (((1,), (1,)), ((), ())),
        preferred_element_type=jnp.float32)

    val = acc + rf_ref[...]
    gmax = jnp.max(val.reshape(_G, 16, val.shape[1]), axis=1)
    val_ref[...] = gmax.T


def _sc_argmax(val_hbm, best_hbm, score_hbm, buf, tmp, bb, sb, *, tpw):
    wid = lax.axis_index("s") * 2 + lax.axis_index("c")
    tbase = wid * tpw
    lane = lax.iota(jnp.int32, 16)

    def chunk_body(c, _):
        pltpu.sync_copy(val_hbm.at[pl.ds(tbase + c * _CW, _CW), :], buf)

        def tok_body(t, m_all):
            m16 = buf[t, pl.ds(0, 16)]
            for g in range(1, _G // 16):
                m16 = jnp.maximum(m16, buf[t, pl.ds(g * 16, 16)])
            for sh in (8, 4, 2, 1):
                tmp[pl.ds(0, 16)] = m16
                tmp[pl.ds(16, 16)] = m16
                m16 = jnp.maximum(m16, tmp[pl.ds(sh, 16)])
            return jnp.where(lane == t, m16, m_all)

        m_all = lax.fori_loop(0, _CW, tok_body,
                              jnp.full((16,), -1.0, jnp.float32))
        cnt = m_all.astype(jnp.int32)
        cf = cnt.astype(jnp.float32)
        k = ((m_all - cf) * float(_P)).astype(jnp.int32)
        bb[pl.ds(c * _CW, _CW)] = (_P - 1) - k
        sb[pl.ds(c * _CW, _CW)] = cf * (1.0 / _D)
        return 0

    lax.fori_loop(0, tpw // _CW, chunk_body, 0)
    pltpu.sync_copy(bb, best_hbm.at[pl.ds(tbase, tpw)])
    pltpu.sync_copy(sb, score_hbm.at[pl.ds(tbase, tpw)])


@functools.lru_cache(maxsize=None)
def _make_sc_argmax(ts):
    tpw = ts // _NW

    @functools.partial(
        pl.kernel,
        mesh=plsc.VectorSubcoreMesh(core_axis_name="c", subcore_axis_name="s"),
        out_type=[
            jax.ShapeDtypeStruct((ts,), jnp.int32),
            jax.ShapeDtypeStruct((ts,), jnp.float32),
        ],
        scratch_types=[
            pltpu.VMEM((_CW, _G), jnp.float32),
            pltpu.VMEM((32,), jnp.float32),
            pltpu.VMEM((tpw,), jnp.int32),
            pltpu.VMEM((tpw,), jnp.float32),
        ],
    )
    def _call(val_hbm, best_hbm, score_hbm, buf, tmp, bb, sb):
        _sc_argmax(val_hbm, best_hbm, score_hbm, buf, tmp, bb, sb, tpw=tpw)

    return _call


def kernel(x, patterns, quantize_edges):
    B, S, D = x.shape
    x2 = x.reshape(B * S, D)
    bests, scores = [], []
    base = 0
    for ts in _SLICES:
        n_steps = ts // _TILE
        step0 = base // _TILE
        val = pl.pallas_call(
            _match_kernel,
            grid=(n_steps,),
            in_specs=[
                pl.BlockSpec(memory_space=pltpu.SMEM),
                pl.BlockSpec((_TILE, D),
                             lambda i, step0=step0: (step0 + i, 0)),
                pl.BlockSpec((_P, D), lambda i: (0, 0)),
            ],
            out_specs=pl.BlockSpec((_TILE, _G), lambda i: (i, 0)),
            out_shape=jax.ShapeDtypeStruct((ts, _G), jnp.float32),
            scratch_shapes=[
                pltpu.VMEM((_P, _N_BINS * _D), jnp.bfloat16),
                pltpu.VMEM((_P, 1), jnp.float32),
            ],
        )(quantize_edges, x2, patterns)
        b1, s1 = _make_sc_argmax(ts)(val)
        bests.append(b1)
        scores.append(s1)
        base += ts
    best = jnp.concatenate(bests)
    score = jnp.concatenate(scores)
    return best.reshape(B, S), score.reshape(B, S)
